# baseline (speedup 1.0000x reference)
import jax
import jax.numpy as jnp
from jax.experimental import pallas as pl
from jax.experimental.pallas import tpu as pltpu

_BB = 2


def _posemb_kernel(x_ref, w_ref, b_ref, p_hbm, o_ref, pvm, psem):
    @pl.when(pl.program_id(0) == 0)
    def _():
        cp = pltpu.make_async_copy(p_hbm, pvm, psem)
        cp.start()
        cp.wait()
        pvm[...] = pvm[...] + b_ref[...]

    for i in range(_BB):
        y = jnp.dot(x_ref[i], w_ref[...], preferred_element_type=jnp.float32)
        o_ref[i] = y + pvm[...]


def kernel(inputs, pos_table, W, b):
    B, L, Din = inputs.shape
    Dout = W.shape[1]
    b2 = b.reshape(1, Dout)
    return pl.pallas_call(
        _posemb_kernel,
        grid=(B // _BB,),
        in_specs=[
            pl.BlockSpec((_BB, L, Din), lambda i: (i, 0, 0)),
            pl.BlockSpec((Din, Dout), lambda i: (0, 0)),
            pl.BlockSpec((1, Dout), lambda i: (0, 0)),
            pl.BlockSpec(memory_space=pltpu.MemorySpace.HBM),
        ],
        out_specs=pl.BlockSpec((_BB, L, Dout), lambda i: (i, 0, 0)),
        out_shape=jax.ShapeDtypeStruct((B, L, Dout), jnp.float32),
        scratch_shapes=[
            pltpu.VMEM((L, Dout), jnp.float32),
            pltpu.SemaphoreType.DMA,
        ],
        compiler_params=pltpu.CompilerParams(
            dimension_semantics=("arbitrary",),
            vmem_limit_bytes=100 * 1024 * 1024,
        ),
    )(inputs, W, b2, pos_table)


# speedup vs baseline: 1.1860x; 1.1860x over previous
import jax
import jax.numpy as jnp
from jax.experimental import pallas as pl
from jax.experimental.pallas import tpu as pltpu

_BB = 2


def _posemb_kernel(x_ref, p_ref, w_ref, b_ref, o_ref):
    pb = p_ref[...] + b_ref[...]
    for i in range(_BB):
        y = jnp.dot(x_ref[i], w_ref[...], preferred_element_type=jnp.float32)
        o_ref[i] = y + pb


def kernel(inputs, pos_table, W, b):
    B, L, Din = inputs.shape
    Dout = W.shape[1]
    b2 = b.reshape(1, Dout)
    return pl.pallas_call(
        _posemb_kernel,
        grid=(B // _BB,),
        in_specs=[
            pl.BlockSpec((_BB, L, Din), lambda i: (i, 0, 0)),
            pl.BlockSpec((L, Dout), lambda i: (0, 0)),
            pl.BlockSpec((Din, Dout), lambda i: (0, 0)),
            pl.BlockSpec((1, Dout), lambda i: (0, 0)),
        ],
        out_specs=pl.BlockSpec((_BB, L, Dout), lambda i: (i, 0, 0)),
        out_shape=jax.ShapeDtypeStruct((B, L, Dout), jnp.float32),
        compiler_params=pltpu.CompilerParams(
            dimension_semantics=("parallel",),
            vmem_limit_bytes=100 * 1024 * 1024,
        ),
    )(inputs, pos_table, W, b2)
